# Initial kernel scaffold; baseline (speedup 1.0000x reference)
import functools

import jax
import jax.numpy as jnp
from jax import lax
from jax.experimental import pallas as pl
from jax.experimental.pallas import tpu as pltpu
from jax.experimental.pallas import tpu_sc as plsc

N = 10000
D = 256
E = 160000

NC = 2
NS = 16
H = N // NC
HP = 5120
RPT = HP // NS
B = 40
EPT = E // NS
NB = EPT // B

_mesh = plsc.VectorSubcoreMesh(core_axis_name="c", subcore_axis_name="s")


def _fill_const(ref, rows, cols, val, dtype):
    v = jnp.full((16,), val, dtype)

    def body(i, _):
        r = i // (cols // 16)
        c = (i % (cols // 16)) * 16
        ref[r, pl.ds(c, 16)] = v
        return 0

    lax.fori_loop(0, rows * (cols // 16), body, 0)


def _make_segsum(want_deg):
    out_types = [jax.ShapeDtypeStruct((N, D), jnp.float32)]
    if want_deg:
        out_types.append(jax.ShapeDtypeStruct((N, 16), jnp.float32))

    @functools.partial(
        pl.kernel,
        mesh=_mesh,
        out_type=tuple(out_types),
        scratch_types=[
            pltpu.VMEM((B,), jnp.int32),
            pltpu.VMEM((B,), jnp.int32),
            pltpu.VMEM((B,), jnp.int32),
            pltpu.VMEM((B, D), jnp.float32),
            pltpu.VMEM((B, 16), jnp.float32),
            pltpu.VMEM((64, D), jnp.float32),
            pltpu.VMEM((64, 16), jnp.float32),
            pltpu.VMEM_SHARED((HP, D), jnp.float32),
            pltpu.VMEM_SHARED((HP, 16), jnp.float32),
            pltpu.SemaphoreType.DMA,
        ],
    )
    def segsum(table, src, dst, *rest):
        if want_deg:
            (agg_out, deg_out, sidx, didx, dstl, rows, ones, zrow, zdeg,
             acc, dega, sem) = rest
        else:
            (agg_out, sidx, didx, dstl, rows, ones, zrow, zdeg,
             acc, dega, sem) = rest
        cid = lax.axis_index("c")
        sid = lax.axis_index("s")
        base_node = cid * H

        _fill_const(zrow, 64, D, 0.0, jnp.float32)
        _fill_const(zdeg, 64, 16, 0.0, jnp.float32)
        if want_deg:
            _fill_const(ones, B, 16, 1.0, jnp.float32)

        for j in range(RPT // 64):
            r0 = sid * RPT + j * 64
            pltpu.sync_copy(zrow, acc.at[pl.ds(r0, 64)])
            if want_deg:
                pltpu.sync_copy(zdeg, dega.at[pl.ds(r0, 64)])
        plsc.subcore_barrier()

        ebase = sid * EPT

        def batch(t, _):
            e0 = ebase + t * B
            pltpu.sync_copy(src.at[pl.ds(e0, B)], sidx)
            pltpu.sync_copy(dst.at[pl.ds(e0, B)], didx)
            gather = pltpu.async_copy(table.at[sidx], rows, sem)
            for k in range(B // 16):
                dv = didx[pl.ds(k * 16, 16)] - base_node
                ok = (dv >= 0) & (dv < H)
                dstl[pl.ds(k * 16, 16)] = jnp.where(ok, dv, H)
            gather.wait()
            pltpu.sync_copy(rows, acc.at[dstl], add=True)
            if want_deg:
                pltpu.sync_copy(ones, dega.at[dstl], add=True)
            return 0

        lax.fori_loop(0, NB, batch, 0)
        plsc.subcore_barrier()

        for j in range(RPT // 64):
            r0 = sid * RPT + j * 64

            @pl.when(r0 < H)
            def _():
                pltpu.sync_copy(acc.at[pl.ds(r0, 64)],
                                agg_out.at[pl.ds(base_node + r0, 64)])
                if want_deg:
                    pltpu.sync_copy(dega.at[pl.ds(r0, 64)],
                                    deg_out.at[pl.ds(base_node + r0, 64)])

    return segsum


_segsum_deg = _make_segsum(True)
_segsum = _make_segsum(False)


@functools.partial(
    pl.kernel,
    mesh=_mesh,
    out_type=(jax.ShapeDtypeStruct((E,), jnp.float32),
              jax.ShapeDtypeStruct((E,), jnp.float32)),
    scratch_types=[
        pltpu.VMEM((B,), jnp.int32),
        pltpu.VMEM((B,), jnp.int32),
        pltpu.VMEM((B, D), jnp.float32),
        pltpu.VMEM((B, D), jnp.float32),
        pltpu.VMEM((B,), jnp.float32),
        pltpu.SemaphoreType.DMA,
        pltpu.SemaphoreType.DMA,
    ],
)
def _edge_dots(table, ps, pd, nsrc, nd, pos_out, neg_out,
               sidx, didx, rows_a, rows_b, res, sem_a, sem_b):
    cid = lax.axis_index("c")
    sid = lax.axis_index("s")
    lanes = lax.iota(jnp.int32, 16)

    def run(src, dst, out):
        ebase = sid * EPT

        def batch(t, _):
            e0 = ebase + t * B
            pltpu.sync_copy(src.at[pl.ds(e0, B)], sidx)
            pltpu.sync_copy(dst.at[pl.ds(e0, B)], didx)
            ga = pltpu.async_copy(table.at[sidx], rows_a, sem_a)
            gb = pltpu.async_copy(table.at[didx], rows_b, sem_b)
            ga.wait()
            gb.wait()
            for g in range(B // 16):
                eids = g * 16 + lanes

                def col(dcol, acc):
                    dv = jnp.full((16,), dcol, jnp.int32)
                    a = plsc.load_gather(rows_a, [eids, dv])
                    bvals = plsc.load_gather(rows_b, [eids, dv])
                    return acc + a * bvals

                accv = lax.fori_loop(0, D, col, jnp.zeros((16,), jnp.float32))
                res[pl.ds(g * 16, 16)] = accv
            pltpu.sync_copy(res, out.at[pl.ds(e0, B)])
            return 0

        lax.fori_loop(0, NB, batch, 0)

    @pl.when(cid == 0)
    def _():
        run(ps, pd, pos_out)

    @pl.when(cid == 1)
    def _():
        run(nsrc, nd, neg_out)


def _make_dense(relu):
    R = 400

    def body(x_ref, agg_ref, deg_ref, ws_ref, wn_ref, b_ref, o_ref):
        deg = jnp.maximum(deg_ref[...][:, 0:1], 1.0)
        mean = agg_ref[...] / deg
        h = (jnp.dot(x_ref[...], ws_ref[...],
                     preferred_element_type=jnp.float32)
             + jnp.dot(mean, wn_ref[...],
                       preferred_element_type=jnp.float32)
             + b_ref[...])
        o_ref[...] = jnp.maximum(h, 0.0) if relu else h

    return pl.pallas_call(
        body,
        grid=(N // R,),
        in_specs=[
            pl.BlockSpec((R, D), lambda i: (i, 0)),
            pl.BlockSpec((R, D), lambda i: (i, 0)),
            pl.BlockSpec((R, 16), lambda i: (i, 0)),
            pl.BlockSpec((D, D), lambda i: (0, 0)),
            pl.BlockSpec((D, D), lambda i: (0, 0)),
            pl.BlockSpec((1, D), lambda i: (0, 0)),
        ],
        out_specs=pl.BlockSpec((R, D), lambda i: (i, 0)),
        out_shape=jax.ShapeDtypeStruct((N, D), jnp.float32),
    )


_dense_relu = _make_dense(True)
_dense_lin = _make_dense(False)


def kernel(x, edge_index, neg_edge_index, W_self, W_neigh, b):
    src = edge_index[0]
    dst = edge_index[1]
    b2d = b.reshape(1, D)
    agg1, deg16 = _segsum_deg(x, src, dst)
    h1 = _dense_relu(x, agg1, deg16, W_self, W_neigh, b2d)
    agg2 = _segsum(h1, src, dst)
    h2 = _dense_lin(h1, agg2, deg16, W_self, W_neigh, b2d)
    pos, neg = _edge_dots(h2, src, dst,
                          neg_edge_index[0], neg_edge_index[1])
    return pos.reshape(E, 1), neg.reshape(E, 1)


# speedup vs baseline: 1.5011x; 1.5011x over previous
import functools

import jax
import jax.numpy as jnp
from jax import lax
from jax.experimental import pallas as pl
from jax.experimental.pallas import tpu as pltpu
from jax.experimental.pallas import tpu_sc as plsc

N = 10000
D = 256
E = 160000

NC = 2
NS = 16
NW = NC * NS
NP = N + 8
OWN = 320
B = 80
CAP = E + 2 * B
CH = 3200
NCH = E // CH
DR = 320
CBUF = 4096
EPT = E // NS
NBS = EPT // B
PACK = 16384

_mesh = plsc.VectorSubcoreMesh(core_axis_name="c", subcore_axis_name="s")


def _fill_const(ref, rows, cols, val, dtype):
    v = jnp.full((16,), val, dtype)

    def body(i, _):
        ref[i // (cols // 16), pl.ds((i % (cols // 16)) * 16, 16)] = v
        return 0

    lax.fori_loop(0, rows * (cols // 16), body, 0)


def _scalar(vec):
    return vec[0]


@functools.partial(
    pl.kernel,
    mesh=_mesh,
    compiler_params=pltpu.CompilerParams(needs_layout_passes=False),
    out_type=(jax.ShapeDtypeStruct((NW * CAP,), jnp.int32),
              jax.ShapeDtypeStruct((NW * 16,), jnp.int32)),
    scratch_types=[
        pltpu.VMEM((CH,), jnp.int32),
        pltpu.VMEM((CH,), jnp.int32),
        pltpu.VMEM((CBUF,), jnp.int32),
        pltpu.VMEM((16,), jnp.int32),
    ],
)
def _compact(src, dst, clist, nbarr, sbuf, dbuf, cbuf, nbuf):
    cid = lax.axis_index("c")
    sid = lax.axis_index("s")
    w = cid * NS + sid
    lo = w * OWN
    hi = jnp.minimum(lo + OWN, N)

    def chunk(c, st):
        cnt, base = st
        e0 = pl.multiple_of(c * CH, 8)
        pltpu.sync_copy(src.at[pl.ds(e0, CH)], sbuf)
        pltpu.sync_copy(dst.at[pl.ds(e0, CH)], dbuf)

        def vec(k, cnt):
            sv = sbuf[pl.ds(k * 16, 16)]
            dv = dbuf[pl.ds(k * 16, 16)]
            own = (dv >= lo) & (dv < hi)
            pk = dv * PACK + sv
            incl = plsc.cumsum(own.astype(jnp.int32))
            plsc.store_scatter(cbuf, [cnt + incl - 1], pk, mask=own)
            return cnt + incl[15]

        cnt = lax.fori_loop(0, CH // 16, vec, cnt)

        nfull = cnt // DR

        def drain(i, _):
            off = i * DR
            dst_off = pl.multiple_of(w * CAP + base + off, 8)
            pltpu.sync_copy(cbuf.at[pl.ds(off, DR)],
                            clist.at[pl.ds(dst_off, DR)])
            return 0

        lax.fori_loop(0, nfull, drain, 0)
        rem = cnt - nfull * DR

        @pl.when(nfull > 0)
        def _():
            def shift(k, _):
                cbuf[pl.ds(k * 16, 16)] = cbuf[pl.ds(nfull * DR + k * 16, 16)]
                return 0

            lax.fori_loop(0, (rem + 15) // 16, shift, 0)

        return (rem, base + nfull * DR)

    cnt, base = lax.fori_loop(0, NCH, chunk, (jnp.int32(0), jnp.int32(0)))

    dummy = jnp.full((16,), N * PACK, jnp.int32)
    for k in range(B // 16 + 1):
        cbuf[pl.ds(cnt + k * 16, 16)] = dummy
    cntr = ((cnt + B - 1) // B) * B

    def flush(i, _):
        off = i * B
        dst_off = pl.multiple_of(w * CAP + base + off, 8)
        pltpu.sync_copy(cbuf.at[pl.ds(off, B)],
                        clist.at[pl.ds(dst_off, B)])
        return 0

    lax.fori_loop(0, cntr // B, flush, 0)
    nb = (base + cntr) // B
    nbuf[pl.ds(0, 16)] = jnp.full((16,), 1, jnp.int32) * nb
    pltpu.sync_copy(nbuf, nbarr.at[pl.ds(pl.multiple_of(w * 16, 8), 16)])


def _make_segsum(want_deg):
    out_types = [jax.ShapeDtypeStruct((NP, D), jnp.float32)]
    if want_deg:
        out_types.append(jax.ShapeDtypeStruct((NP, D), jnp.float32))

    @functools.partial(
        pl.kernel,
        mesh=_mesh,
        compiler_params=pltpu.CompilerParams(
            needs_layout_passes=False, use_tc_tiling_on_sc=False),
        out_type=tuple(out_types) if want_deg else out_types[0],
        scratch_types=[
            pltpu.VMEM((B + 16,), jnp.int32),
            pltpu.VMEM((B,), jnp.int32),
            pltpu.VMEM((B, D // 2), jnp.int32),
            pltpu.VMEM((OWN + 8, D), jnp.float32),
            pltpu.VMEM((40, D), jnp.float32),
            pltpu.VMEM((16,), jnp.int32),
            pltpu.SMEM((OWN + 8,), jnp.int32),
            pltpu.SemaphoreType.DMA,
            pltpu.SemaphoreType.DMA,
        ],
    )
    def segsum(table, clist, nbarr, *rest):
        if want_deg:
            (agg, deg, pbuf, sidx, rows, acc, drow,
             nbuf, dsm, sem, sem2) = rest
        else:
            (agg, pbuf, sidx, rows, acc, drow,
             nbuf, dsm, sem, sem2) = rest
        cid = lax.axis_index("c")
        sid = lax.axis_index("s")
        w = cid * NS + sid
        lo = w * OWN
        hi = jnp.minimum(lo + OWN, N)

        _fill_const(acc, OWN + 8, D, 0.0, jnp.float32)
        if want_deg:
            def zdeg(i, _):
                dsm[i] = 0
                return 0

            lax.fori_loop(0, OWN + 8, zdeg, 0)

        pltpu.sync_copy(nbarr.at[pl.ds(pl.multiple_of(w * 16, 8), 16)], nbuf)
        nb = _scalar(nbuf[pl.ds(0, 16)])

        def batch(t, _):
            off = pl.multiple_of(w * CAP + t * B, 8)
            pltpu.sync_copy(clist.at[pl.ds(off, B)], pbuf.at[pl.ds(0, B)])
            for k in range(B // 16):
                v = pbuf[pl.ds(k * 16, 16)]
                sidx[pl.ds(k * 16, 16)] = v - (v // PACK) * PACK
            pltpu.async_copy(table.at[sidx], rows, sem).wait()

            himask = jnp.full((16,), -65536, jnp.int32)

            def edge(e, _):
                pk0 = pbuf[pl.ds(e, 16)][0]
                dl = jnp.minimum(pk0 // PACK - lo, OWN)
                for c in range(D // 32):
                    cs = pl.ds(c * 16, 16)
                    ch = pl.ds(D // 2 + c * 16, 16)
                    w = rows[e, cs]
                    acc[dl, cs] = acc[dl, cs] + plsc.bitcast(
                        w << 16, jnp.float32)
                    acc[dl, ch] = acc[dl, ch] + plsc.bitcast(
                        w & himask, jnp.float32)
                if want_deg:
                    dsm[dl] = dsm[dl] + 1
                return 0

            lax.fori_loop(0, B, edge, 0)
            return 0

        lax.fori_loop(0, nb, batch, 0)

        for j in range(OWN // 40):
            r0 = pl.multiple_of(lo + j * 40, 8)

            @pl.when(r0 < hi)
            def _():
                pltpu.sync_copy(acc.at[pl.ds(j * 40, 40)],
                                agg.at[pl.ds(r0, 40)])

        if want_deg:
            for j in range(OWN // 40):
                r0 = pl.multiple_of(lo + j * 40, 8)

                @pl.when(r0 < hi)
                def _():
                    def fill_deg(i, _):
                        val = (dsm[j * 40 + i]).astype(jnp.float32)
                        vv = jnp.full((16,), 1.0, jnp.float32) * val
                        for c in range(D // 16):
                            drow[i, pl.ds(c * 16, 16)] = vv
                        return 0

                    lax.fori_loop(0, 40, fill_deg, 0)
                    pltpu.sync_copy(drow, deg.at[pl.ds(r0, 40)])

    return segsum


_segsum_deg = _make_segsum(True)
_segsum = _make_segsum(False)


@functools.partial(
    pl.kernel,
    mesh=_mesh,
    compiler_params=pltpu.CompilerParams(
        needs_layout_passes=False, use_tc_tiling_on_sc=False),
    out_type=(jax.ShapeDtypeStruct((E,), jnp.float32),
              jax.ShapeDtypeStruct((E,), jnp.float32)),
    scratch_types=[
        pltpu.VMEM((B,), jnp.int32),
        pltpu.VMEM((B,), jnp.int32),
        pltpu.VMEM((B,), jnp.int32),
        pltpu.VMEM((B,), jnp.int32),
        pltpu.VMEM((B, D // 2), jnp.int32),
        pltpu.VMEM((B, D // 2), jnp.int32),
        pltpu.VMEM((B, D // 2), jnp.int32),
        pltpu.VMEM((B, D // 2), jnp.int32),
        pltpu.VMEM((B,), jnp.float32),
        pltpu.SemaphoreType.DMA,
        pltpu.SemaphoreType.DMA,
        pltpu.SemaphoreType.DMA,
        pltpu.SemaphoreType.DMA,
    ],
)
def _edge_dots(table, ps, pd, nsv, ndv, pos_out, neg_out,
               sidx0, didx0, sidx1, didx1, ra0, rb0, ra1, rb1,
               res, sa0, sb0, sa1, sb1):
    cid = lax.axis_index("c")
    sid = lax.axis_index("s")
    lanes = lax.iota(jnp.int32, 16)
    bufs = ((sidx0, didx0, ra0, rb0, sa0, sb0),
            (sidx1, didx1, ra1, rb1, sa1, sb1))

    def run(srcr, dstr, out):
        ebase = sid * EPT

        def load(t, bi):
            sidx, didx, ra, rb, sa, sb = bufs[bi]
            e0 = pl.multiple_of(ebase + t * B, 8)
            pltpu.sync_copy(srcr.at[pl.ds(e0, B)], sidx)
            pltpu.sync_copy(dstr.at[pl.ds(e0, B)], didx)
            pltpu.async_copy(table.at[sidx], ra, sa)
            pltpu.async_copy(table.at[didx], rb, sb)

        def compute(t, bi):
            sidx, didx, ra, rb, sa, sb = bufs[bi]
            pltpu.make_async_copy(table.at[sidx], ra, sa).wait()
            pltpu.make_async_copy(table.at[didx], rb, sb).wait()
            for g in range(B // 16):
                eids = g * 16 + lanes

                himask = jnp.full((16,), -65536, jnp.int32)

                def col16(d2, acc):
                    for cc in range(16):
                        dv = jnp.full((16,), 16, jnp.int32) * d2 + cc
                        aw = plsc.load_gather(ra, [eids, dv])
                        bw = plsc.load_gather(rb, [eids, dv])
                        alo = plsc.bitcast(aw << 16, jnp.float32)
                        blo = plsc.bitcast(bw << 16, jnp.float32)
                        ahi = plsc.bitcast(aw & himask, jnp.float32)
                        bhi = plsc.bitcast(bw & himask, jnp.float32)
                        acc = acc + alo * blo + ahi * bhi
                    return acc

                accv = lax.fori_loop(0, D // 32, col16,
                                     jnp.zeros((16,), jnp.float32))
                res[pl.ds(g * 16, 16)] = accv
            e0 = pl.multiple_of(ebase + t * B, 8)
            pltpu.sync_copy(res, out.at[pl.ds(e0, B)])

        load(0, 0)

        def pair(p, _):
            for b in range(2):
                t = p * 2 + b
                load(t + 1, 1 - b)
                compute(t, b)
            return 0

        lax.fori_loop(0, (NBS - 1) // 2, pair, 0)
        compute(NBS - 1, (NBS - 1) % 2)

    @pl.when(cid == 0)
    def _():
        run(ps, pd, pos_out)

    @pl.when(cid == 1)
    def _():
        run(nsv, ndv, neg_out)


def _make_dense(relu):
    R = 400

    def body(x_ref, agg_ref, deg_ref, ws_ref, wn_ref, b_ref, o_ref):
        deg = jnp.maximum(deg_ref[...][:, 0:1], 1.0)
        mean = agg_ref[...] / deg
        h = (jnp.dot(x_ref[...], ws_ref[...],
                     preferred_element_type=jnp.float32)
             + jnp.dot(mean, wn_ref[...],
                       preferred_element_type=jnp.float32)
             + b_ref[...])
        o_ref[...] = jnp.maximum(h, 0.0) if relu else h

    return pl.pallas_call(
        body,
        grid=(N // R,),
        in_specs=[
            pl.BlockSpec((R, D), lambda i: (i, 0)),
            pl.BlockSpec((R, D), lambda i: (i, 0)),
            pl.BlockSpec((R, D), lambda i: (i, 0)),
            pl.BlockSpec((D, D), lambda i: (0, 0)),
            pl.BlockSpec((D, D), lambda i: (0, 0)),
            pl.BlockSpec((1, D), lambda i: (0, 0)),
        ],
        out_specs=pl.BlockSpec((R, D), lambda i: (i, 0)),
        out_shape=jax.ShapeDtypeStruct((N, D), jnp.float32),
    )


_dense_relu = _make_dense(True)
_dense_lin = _make_dense(False)


def _packbf(h):
    return jax.lax.bitcast_convert_type(
        h.astype(jnp.bfloat16).reshape(h.shape[0], D // 2, 2), jnp.int32)


def kernel(x, edge_index, neg_edge_index, W_self, W_neigh, b):
    src = edge_index[0]
    dst = edge_index[1]
    b2d = b.reshape(1, D)
    Wn_p = jnp.concatenate([W_neigh[0::2], W_neigh[1::2]], axis=0)
    clist, nbarr = _compact(src, dst)
    agg1, deg16 = _segsum_deg(_packbf(x), clist, nbarr)
    h1 = _dense_relu(x, agg1, deg16, W_self, Wn_p, b2d)
    agg2 = _segsum(_packbf(h1), clist, nbarr)
    h2 = _dense_lin(h1, agg2, deg16, W_self, Wn_p, b2d)
    h2p = jax.lax.bitcast_convert_type(
        h2.astype(jnp.bfloat16).reshape(N, D // 2, 2), jnp.int32)
    pos, neg = _edge_dots(h2p, src, dst,
                          neg_edge_index[0], neg_edge_index[1])
    return pos.reshape(E, 1), neg.reshape(E, 1)
